# baseline (speedup 1.0000x reference)
import functools

import jax
import jax.numpy as jnp
from jax import lax
from jax.experimental import pallas as pl
from jax.experimental.pallas import tpu as pltpu
from jax.experimental.pallas import tpu_sc as plsc

_N = 10000
_NPAD = 10240
_TRASH = _N
_NC, _NS = 2, 16
_NW = _NC * _NS
_K = 128
_TW = 128
_NACC = 10112
_BLK = 256
_NB = _NPAD // _BLK


def _emit_tables(xx, w_ref, as_ref, ad_ref, t_ref, a_ref, *, hw, nheads):
    h = jnp.dot(xx, w_ref[...], precision=lax.Precision.HIGHEST,
                preferred_element_type=jnp.float32)
    g = hw // nheads
    ri = lax.broadcasted_iota(jnp.int32, (hw, hw), 0) // g
    ci = lax.broadcasted_iota(jnp.int32, (hw, hw), 1) // g
    gm = (ri == ci).astype(jnp.float32)
    asn = jnp.dot(h * as_ref[0:1, :], gm, precision=lax.Precision.HIGHEST,
                  preferred_element_type=jnp.float32)
    adn = jnp.dot(h * ad_ref[0:1, :], gm, precision=lax.Precision.HIGHEST,
                  preferred_element_type=jnp.float32)
    tparts = [h, asn]
    if _TW > 2 * hw:
        tparts.append(jnp.zeros((h.shape[0], _TW - 2 * hw), jnp.float32))
    aparts = [adn]
    if _TW > hw:
        aparts.append(jnp.zeros((h.shape[0], _TW - hw), jnp.float32))
    t_ref[...] = jnp.concatenate(tparts, axis=1)
    a_ref[...] = jnp.concatenate(aparts, axis=1)


def _front_body(x_ref, w_ref, as_ref, ad_ref, t_ref, a_ref, *, hw, nheads):
    _emit_tables(x_ref[...], w_ref, as_ref, ad_ref, t_ref, a_ref,
                 hw=hw, nheads=nheads)


def _combine_front_body(p0_ref, p1_ref, b_ref, w_ref, as_ref, ad_ref,
                        t_ref, a_ref, *, hw_prev, hw, nheads):
    acc = p0_ref[...] + p1_ref[...]
    num = acc[:, :hw_prev]
    den = acc[:, hw_prev:2 * hw_prev] + 1e-16
    xx = num / den + b_ref[0:1, :]
    xx = jnp.where(xx > 0, xx, jnp.exp(jnp.minimum(xx, 0.0)) - 1.0)
    _emit_tables(xx, w_ref, as_ref, ad_ref, t_ref, a_ref, hw=hw, nheads=nheads)


def _final_body(p0_ref, p1_ref, b_ref, batch_ref, lw_ref, lb_ref, out_ref,
                sums_ref, cnt_ref, *, hw_prev, nblocks):
    i = pl.program_id(0)

    @pl.when(i == 0)
    def _init():
        sums_ref[...] = jnp.zeros_like(sums_ref)
        cnt_ref[...] = jnp.zeros_like(cnt_ref)

    acc = p0_ref[...] + p1_ref[...]
    num = acc[:, :hw_prev]
    den = acc[:, hw_prev:2 * hw_prev] + 1e-16
    hb = num / den + b_ref[0:1, :]
    bidx = batch_ref[...]
    gi = lax.broadcasted_iota(jnp.int32, (1, 64), 1)
    oh = (bidx == gi).astype(jnp.float32)
    sums_ref[...] += lax.dot_general(
        oh, hb, (((0,), (0,)), ((), ())),
        precision=lax.Precision.HIGHEST, preferred_element_type=jnp.float32)
    cnt_ref[...] += lax.dot_general(
        oh, jnp.ones((hb.shape[0], hw_prev), jnp.float32),
        (((0,), (0,)), ((), ())),
        precision=lax.Precision.HIGHEST, preferred_element_type=jnp.float32)

    @pl.when(i == nblocks - 1)
    def _fin():
        pooled = sums_ref[...] / jnp.maximum(cnt_ref[...], 1.0)
        out_ref[...] = jnp.dot(pooled, lw_ref[...],
                               precision=lax.Precision.HIGHEST,
                               preferred_element_type=jnp.float32) + lb_ref[0:1, :]


def _front_call(fin, hw, nheads):
    body = functools.partial(_front_body, hw=hw, nheads=nheads)
    return pl.pallas_call(
        body,
        grid=(_NB,),
        in_specs=[
            pl.BlockSpec((_BLK, fin), lambda i: (i, 0)),
            pl.BlockSpec((fin, hw), lambda i: (0, 0)),
            pl.BlockSpec((8, hw), lambda i: (0, 0)),
            pl.BlockSpec((8, hw), lambda i: (0, 0)),
        ],
        out_specs=[
            pl.BlockSpec((_BLK, _TW), lambda i: (i, 0)),
            pl.BlockSpec((_BLK, _TW), lambda i: (i, 0)),
        ],
        out_shape=[
            jax.ShapeDtypeStruct((_NPAD, _TW), jnp.float32),
            jax.ShapeDtypeStruct((_NPAD, _TW), jnp.float32),
        ],
    )


def _combine_front_call(hw_prev, hw, nheads):
    body = functools.partial(_combine_front_body, hw_prev=hw_prev,
                             hw=hw, nheads=nheads)
    return pl.pallas_call(
        body,
        grid=(_NB,),
        in_specs=[
            pl.BlockSpec((_BLK, _TW), lambda i: (i, 0)),
            pl.BlockSpec((_BLK, _TW), lambda i: (i, 0)),
            pl.BlockSpec((8, hw_prev), lambda i: (0, 0)),
            pl.BlockSpec((hw_prev, hw), lambda i: (0, 0)),
            pl.BlockSpec((8, hw), lambda i: (0, 0)),
            pl.BlockSpec((8, hw), lambda i: (0, 0)),
        ],
        out_specs=[
            pl.BlockSpec((_BLK, _TW), lambda i: (i, 0)),
            pl.BlockSpec((_BLK, _TW), lambda i: (i, 0)),
        ],
        out_shape=[
            jax.ShapeDtypeStruct((_NPAD, _TW), jnp.float32),
            jax.ShapeDtypeStruct((_NPAD, _TW), jnp.float32),
        ],
    )


def _final_call(hw_prev):
    body = functools.partial(_final_body, hw_prev=hw_prev, nblocks=_NB)
    return pl.pallas_call(
        body,
        grid=(_NB,),
        in_specs=[
            pl.BlockSpec((_BLK, _TW), lambda i: (i, 0)),
            pl.BlockSpec((_BLK, _TW), lambda i: (i, 0)),
            pl.BlockSpec((8, hw_prev), lambda i: (0, 0)),
            pl.BlockSpec((_BLK, 1), lambda i: (i, 0)),
            pl.BlockSpec((hw_prev, 128), lambda i: (0, 0)),
            pl.BlockSpec((8, 128), lambda i: (0, 0)),
        ],
        out_specs=pl.BlockSpec((64, 128), lambda i: (0, 0)),
        out_shape=jax.ShapeDtypeStruct((64, 128), jnp.float32),
        scratch_shapes=[
            pltpu.VMEM((64, hw_prev), jnp.float32),
            pltpu.VMEM((64, hw_prev), jnp.float32),
        ],
    )


def _edge_body(t_hbm, a_hbm, src_hbm, dst_hbm, out0, out1,
               src_v, dst_v, srow_v, arow_v, vals_v, acc_sh, sem0, sem1,
               *, hw, nheads, pw, nchunks):
    cid = lax.axis_index("c")
    sid = lax.axis_index("s")
    wid = sid * _NC + cid
    tile_rows = _NACC // _NS
    chunks = []
    off = 0
    while off < tile_rows:
        step = min(_K, tile_rows - off)
        chunks.append((off, step))
        off += step

    def _zrow(i, c):
        for v in range(_TW // 16):
            vals_v[i, pl.ds(16 * v, 16)] = jnp.zeros((16,), jnp.float32)
        return c
    lax.fori_loop(0, _K, _zrow, 0)

    for off, step in chunks:
        pltpu.sync_copy(vals_v.at[pl.ds(0, step)],
                        acc_sh.at[pl.ds(sid * tile_rows + off, step)])
    plsc.subcore_barrier()

    def _chunk(ci_, c):
        base = wid * pw + ci_ * _K
        pltpu.sync_copy(src_hbm.at[pl.ds(base, _K)], src_v)
        pltpu.sync_copy(dst_hbm.at[pl.ds(base, _K)], dst_v)
        cp0 = pltpu.async_copy(t_hbm.at[src_v], srow_v, sem0)
        cp1 = pltpu.async_copy(a_hbm.at[dst_v], arow_v, sem1)
        cp0.wait()
        cp1.wait()

        def _edge(kk4, cc):
            for u in range(4):
                kk = kk4 * 4 + u
                for v in range(hw // 16):
                    a_s = srow_v[kk, pl.ds(hw + 16 * v, 16)]
                    a_d = arow_v[kk, pl.ds(16 * v, 16)]
                    al = a_s + a_d
                    al = jnp.maximum(al, al * 0.2)
                    e = jnp.exp(al)
                    h_s = srow_v[kk, pl.ds(16 * v, 16)]
                    vals_v[kk, pl.ds(16 * v, 16)] = h_s * e
                    vals_v[kk, pl.ds(hw + 16 * v, 16)] = e
            return cc
        lax.fori_loop(0, _K // 4, _edge, 0)

        pltpu.sync_copy(vals_v, acc_sh.at[dst_v], add=True)
        return c
    lax.fori_loop(0, nchunks, _chunk, 0)
    plsc.subcore_barrier()

    for off, step in chunks:
        row = sid * tile_rows + off

        @pl.when(cid == 0)
        def _w0():
            pltpu.sync_copy(acc_sh.at[pl.ds(row, step)],
                            out0.at[pl.ds(row, step)])

        @pl.when(cid == 1)
        def _w1():
            pltpu.sync_copy(acc_sh.at[pl.ds(row, step)],
                            out1.at[pl.ds(row, step)])

    @pl.when(sid == _NS - 1)
    def _tail():
        extra = _NPAD - _NACC

        @pl.when(cid == 0)
        def _t0():
            pltpu.sync_copy(acc_sh.at[pl.ds(0, extra)],
                            out0.at[pl.ds(_NACC, extra)])

        @pl.when(cid == 1)
        def _t1():
            pltpu.sync_copy(acc_sh.at[pl.ds(0, extra)],
                            out1.at[pl.ds(_NACC, extra)])


def _edge_call(hw, nheads, e2):
    pw = e2 // _NW
    nchunks = pw // _K
    body = functools.partial(_edge_body, hw=hw, nheads=nheads, pw=pw,
                             nchunks=nchunks)
    mesh = plsc.VectorSubcoreMesh(core_axis_name="c", subcore_axis_name="s")
    return functools.partial(
        pl.kernel,
        mesh=mesh,
        out_type=[
            jax.ShapeDtypeStruct((_NPAD, _TW), jnp.float32),
            jax.ShapeDtypeStruct((_NPAD, _TW), jnp.float32),
        ],
        scratch_types=[
            pltpu.VMEM((_K,), jnp.int32),
            pltpu.VMEM((_K,), jnp.int32),
            pltpu.VMEM((_K, _TW), jnp.float32),
            pltpu.VMEM((_K, _TW), jnp.float32),
            pltpu.VMEM((_K, _TW), jnp.float32),
            pltpu.VMEM_SHARED((_NACC, _TW), jnp.float32),
            pltpu.SemaphoreType.DMA,
            pltpu.SemaphoreType.DMA,
        ],
    )(body)


def _row8(v):
    return jnp.broadcast_to(v.reshape(1, -1), (8, v.size))


def kernel(x, edge_index, batch, W1, as1, ad1, b1, W2, as2, ad2, b2,
           W3, as3, ad3, b3, lW, lb):
    n = x.shape[0]
    e_tot = edge_index.shape[1] + n
    unit = _NW * _K * 2
    e2 = ((e_tot + unit - 1) // unit) * unit
    epad = e2 - e_tot

    loop = jnp.arange(n, dtype=jnp.int32)
    src = jnp.concatenate(
        [edge_index[0].astype(jnp.int32), loop,
         jnp.zeros((epad,), jnp.int32)])
    dst = jnp.concatenate(
        [edge_index[1].astype(jnp.int32), loop,
         jnp.full((epad,), _TRASH, jnp.int32)])

    xp = jnp.pad(x, ((0, _NPAD - n), (0, 0)))
    batch2d = jnp.pad(batch.astype(jnp.int32), (0, _NPAD - n),
                      constant_values=10000).reshape(_NPAD, 1)
    lwp = jnp.pad(lW, ((0, 0), (0, 128 - lW.shape[1])))
    lbp = _row8(jnp.pad(lb, (0, 128 - lb.shape[0])))

    t1, a1 = _front_call(128, 64, 8)(
        xp, W1, _row8(as1.reshape(-1)), _row8(ad1.reshape(-1)))
    p0, p1 = _edge_call(64, 8, e2)(t1, a1, src, dst)

    t2, a2 = _combine_front_call(64, 64, 8)(
        p0, p1, _row8(b1), W2, _row8(as2.reshape(-1)), _row8(ad2.reshape(-1)))
    q0, q1 = _edge_call(64, 8, e2)(t2, a2, src, dst)

    t3, a3 = _combine_front_call(64, 32, 1)(
        q0, q1, _row8(b2), W3, _row8(as3.reshape(-1)), _row8(ad3.reshape(-1)))
    r0, r1 = _edge_call(32, 1, e2)(t3, a3, src, dst)

    out = _final_call(32)(r0, r1, _row8(b3), batch2d, lwp, lbp)
    return out[:64, :2]


# speedup vs baseline: 1.9017x; 1.9017x over previous
import functools

import jax
import jax.numpy as jnp
from jax import lax
from jax.experimental import pallas as pl
from jax.experimental.pallas import tpu as pltpu
from jax.experimental.pallas import tpu_sc as plsc

_N = 10000
_NPAD = 10240
_TRASH = _N
_NC, _NS = 2, 16
_NW = _NC * _NS
_K = 64
_TW = 128
_NACC = 10112
_BLK = 256
_NB = _NPAD // _BLK


def _emit_tables(xx, w_ref, as_ref, ad_ref, t_ref, a_ref, *, hw, nheads):
    h = jnp.dot(xx, w_ref[...], precision=lax.Precision.HIGHEST,
                preferred_element_type=jnp.float32)
    g = hw // nheads
    ri = lax.broadcasted_iota(jnp.int32, (hw, hw), 0) // g
    ci = lax.broadcasted_iota(jnp.int32, (hw, hw), 1) // g
    gm = (ri == ci).astype(jnp.float32)
    asn = jnp.dot(h * as_ref[0:1, :], gm, precision=lax.Precision.HIGHEST,
                  preferred_element_type=jnp.float32)
    adn = jnp.dot(h * ad_ref[0:1, :], gm, precision=lax.Precision.HIGHEST,
                  preferred_element_type=jnp.float32)
    tparts = [h, asn]
    if _TW > 2 * hw:
        tparts.append(jnp.zeros((h.shape[0], _TW - 2 * hw), jnp.float32))
    aparts = [adn]
    if _TW > hw:
        aparts.append(jnp.zeros((h.shape[0], _TW - hw), jnp.float32))
    t_ref[...] = jnp.concatenate(tparts, axis=1)
    a_ref[...] = jnp.concatenate(aparts, axis=1)


def _front_body(x_ref, w_ref, as_ref, ad_ref, t_ref, a_ref, *, hw, nheads):
    _emit_tables(x_ref[...], w_ref, as_ref, ad_ref, t_ref, a_ref,
                 hw=hw, nheads=nheads)


def _combine_front_body(p0_ref, p1_ref, b_ref, w_ref, as_ref, ad_ref,
                        t_ref, a_ref, *, hw_prev, hw, nheads):
    acc = p0_ref[...] + p1_ref[...]
    num = acc[:, :hw_prev]
    den = acc[:, hw_prev:2 * hw_prev] + 1e-16
    xx = num / den + b_ref[0:1, :]
    xx = jnp.where(xx > 0, xx, jnp.exp(jnp.minimum(xx, 0.0)) - 1.0)
    _emit_tables(xx, w_ref, as_ref, ad_ref, t_ref, a_ref, hw=hw, nheads=nheads)


def _final_body(p0_ref, p1_ref, b_ref, batch_ref, lw_ref, lb_ref, out_ref,
                sums_ref, cnt_ref, *, hw_prev, nblocks):
    i = pl.program_id(0)

    @pl.when(i == 0)
    def _init():
        sums_ref[...] = jnp.zeros_like(sums_ref)
        cnt_ref[...] = jnp.zeros_like(cnt_ref)

    acc = p0_ref[...] + p1_ref[...]
    num = acc[:, :hw_prev]
    den = acc[:, hw_prev:2 * hw_prev] + 1e-16
    hb = num / den + b_ref[0:1, :]
    bidx = batch_ref[...]
    gi = lax.broadcasted_iota(jnp.int32, (1, 64), 1)
    oh = (bidx == gi).astype(jnp.float32)
    sums_ref[...] += lax.dot_general(
        oh, hb, (((0,), (0,)), ((), ())),
        precision=lax.Precision.HIGHEST, preferred_element_type=jnp.float32)
    cnt_ref[...] += lax.dot_general(
        oh, jnp.ones((hb.shape[0], hw_prev), jnp.float32),
        (((0,), (0,)), ((), ())),
        precision=lax.Precision.HIGHEST, preferred_element_type=jnp.float32)

    @pl.when(i == nblocks - 1)
    def _fin():
        pooled = sums_ref[...] / jnp.maximum(cnt_ref[...], 1.0)
        out_ref[...] = jnp.dot(pooled, lw_ref[...],
                               precision=lax.Precision.HIGHEST,
                               preferred_element_type=jnp.float32) + lb_ref[0:1, :]


def _front_call(fin, hw, nheads):
    body = functools.partial(_front_body, hw=hw, nheads=nheads)
    return pl.pallas_call(
        body,
        grid=(_NB,),
        in_specs=[
            pl.BlockSpec((_BLK, fin), lambda i: (i, 0)),
            pl.BlockSpec((fin, hw), lambda i: (0, 0)),
            pl.BlockSpec((8, hw), lambda i: (0, 0)),
            pl.BlockSpec((8, hw), lambda i: (0, 0)),
        ],
        out_specs=[
            pl.BlockSpec((_BLK, _TW), lambda i: (i, 0)),
            pl.BlockSpec((_BLK, _TW), lambda i: (i, 0)),
        ],
        out_shape=[
            jax.ShapeDtypeStruct((_NPAD, _TW), jnp.float32),
            jax.ShapeDtypeStruct((_NPAD, _TW), jnp.float32),
        ],
    )


def _combine_front_call(hw_prev, hw, nheads):
    body = functools.partial(_combine_front_body, hw_prev=hw_prev,
                             hw=hw, nheads=nheads)
    return pl.pallas_call(
        body,
        grid=(_NB,),
        in_specs=[
            pl.BlockSpec((_BLK, _TW), lambda i: (i, 0)),
            pl.BlockSpec((_BLK, _TW), lambda i: (i, 0)),
            pl.BlockSpec((8, hw_prev), lambda i: (0, 0)),
            pl.BlockSpec((hw_prev, hw), lambda i: (0, 0)),
            pl.BlockSpec((8, hw), lambda i: (0, 0)),
            pl.BlockSpec((8, hw), lambda i: (0, 0)),
        ],
        out_specs=[
            pl.BlockSpec((_BLK, _TW), lambda i: (i, 0)),
            pl.BlockSpec((_BLK, _TW), lambda i: (i, 0)),
        ],
        out_shape=[
            jax.ShapeDtypeStruct((_NPAD, _TW), jnp.float32),
            jax.ShapeDtypeStruct((_NPAD, _TW), jnp.float32),
        ],
    )


def _final_call(hw_prev):
    body = functools.partial(_final_body, hw_prev=hw_prev, nblocks=_NB)
    return pl.pallas_call(
        body,
        grid=(_NB,),
        in_specs=[
            pl.BlockSpec((_BLK, _TW), lambda i: (i, 0)),
            pl.BlockSpec((_BLK, _TW), lambda i: (i, 0)),
            pl.BlockSpec((8, hw_prev), lambda i: (0, 0)),
            pl.BlockSpec((_BLK, 1), lambda i: (i, 0)),
            pl.BlockSpec((hw_prev, 128), lambda i: (0, 0)),
            pl.BlockSpec((8, 128), lambda i: (0, 0)),
        ],
        out_specs=pl.BlockSpec((64, 128), lambda i: (0, 0)),
        out_shape=jax.ShapeDtypeStruct((64, 128), jnp.float32),
        scratch_shapes=[
            pltpu.VMEM((64, hw_prev), jnp.float32),
            pltpu.VMEM((64, hw_prev), jnp.float32),
        ],
    )


def _edge_body(t_hbm, a_hbm, sd_hbm, out0, out1,
               idx0_v, idx1_v, srow0_v, srow1_v, arow0_v, arow1_v,
               vals_v, acc_sh, st0, st1, sa0, sa1,
               *, hw, nheads, pw, nchunks):
    cid = lax.axis_index("c")
    sid = lax.axis_index("s")
    wid = sid * _NC + cid
    tile_rows = _NACC // _NS
    chunks = []
    off = 0
    while off < tile_rows:
        step = min(_K, tile_rows - off)
        chunks.append((off, step))
        off += step

    def _zrow(i, c):
        for v in range(_TW // 16):
            vals_v[i, pl.ds(16 * v, 16)] = jnp.zeros((16,), jnp.float32)
        return c
    lax.fori_loop(0, _K, _zrow, 0)

    for off, step in chunks:
        pltpu.sync_copy(vals_v.at[pl.ds(0, step)],
                        acc_sh.at[pl.ds(sid * tile_rows + off, step)])
    plsc.subcore_barrier()

    bufs = ((idx0_v, srow0_v, arow0_v, st0, sa0),
            (idx1_v, srow1_v, arow1_v, st1, sa1))
    npairs = nchunks // 2

    def _start(ci_, b):
        idx_v, srow_v, arow_v, st, sa = bufs[b]
        base2 = (wid * nchunks + ci_) * 2 * _K
        pltpu.sync_copy(sd_hbm.at[pl.ds(base2, 2 * _K)], idx_v)
        pltpu.async_copy(t_hbm.at[idx_v.at[pl.ds(0, _K)]], srow_v, st)
        pltpu.async_copy(a_hbm.at[idx_v.at[pl.ds(_K, _K)]], arow_v, sa)

    _start(0, 0)
    _start(1, 1)

    def _pair(p, c):
        for b in range(2):
            ci_ = 2 * p + b
            idx_v, srow_v, arow_v, st, sa = bufs[b]
            pltpu.make_async_copy(
                t_hbm.at[idx_v.at[pl.ds(0, _K)]], srow_v, st).wait()
            pltpu.make_async_copy(
                a_hbm.at[idx_v.at[pl.ds(_K, _K)]], arow_v, sa).wait()

            def _edge(kk, cc):
                for v in range(hw // 16):
                    a_s = srow_v[kk, pl.ds(hw + 16 * v, 16)]
                    a_d = arow_v[kk, pl.ds(16 * v, 16)]
                    al = a_s + a_d
                    al = jnp.maximum(al, al * 0.2)
                    e = jnp.exp(al)
                    h_s = srow_v[kk, pl.ds(16 * v, 16)]
                    vals_v[kk, pl.ds(16 * v, 16)] = h_s * e
                    vals_v[kk, pl.ds(hw + 16 * v, 16)] = e
                return cc
            lax.fori_loop(0, _K, _edge, 0)

            pltpu.sync_copy(vals_v, acc_sh.at[idx_v.at[pl.ds(_K, _K)]],
                            add=True)

            @pl.when(p + 1 < npairs)
            def _pf():
                _start(ci_ + 2, b)
        return c
    lax.fori_loop(0, npairs, _pair, 0)
    plsc.subcore_barrier()

    for off, step in chunks:
        row = sid * tile_rows + off

        @pl.when(cid == 0)
        def _w0():
            pltpu.sync_copy(acc_sh.at[pl.ds(row, step)],
                            out0.at[pl.ds(row, step)])

        @pl.when(cid == 1)
        def _w1():
            pltpu.sync_copy(acc_sh.at[pl.ds(row, step)],
                            out1.at[pl.ds(row, step)])

    @pl.when(sid == _NS - 1)
    def _tail():
        extra = _NPAD - _NACC

        @pl.when(cid == 0)
        def _t0():
            pltpu.sync_copy(acc_sh.at[pl.ds(0, extra)],
                            out0.at[pl.ds(_NACC, extra)])

        @pl.when(cid == 1)
        def _t1():
            pltpu.sync_copy(acc_sh.at[pl.ds(0, extra)],
                            out1.at[pl.ds(_NACC, extra)])


def _edge_call(hw, nheads, e2):
    pw = e2 // _NW
    nchunks = pw // _K
    body = functools.partial(_edge_body, hw=hw, nheads=nheads, pw=pw,
                             nchunks=nchunks)
    mesh = plsc.VectorSubcoreMesh(core_axis_name="c", subcore_axis_name="s")
    return functools.partial(
        pl.kernel,
        mesh=mesh,
        out_type=[
            jax.ShapeDtypeStruct((_NPAD, _TW), jnp.float32),
            jax.ShapeDtypeStruct((_NPAD, _TW), jnp.float32),
        ],
        scratch_types=[
            pltpu.VMEM((2 * _K,), jnp.int32),
            pltpu.VMEM((2 * _K,), jnp.int32),
            pltpu.VMEM((_K, _TW), jnp.float32),
            pltpu.VMEM((_K, _TW), jnp.float32),
            pltpu.VMEM((_K, _TW), jnp.float32),
            pltpu.VMEM((_K, _TW), jnp.float32),
            pltpu.VMEM((_K, _TW), jnp.float32),
            pltpu.VMEM_SHARED((_NACC, _TW), jnp.float32),
            pltpu.SemaphoreType.DMA,
            pltpu.SemaphoreType.DMA,
            pltpu.SemaphoreType.DMA,
            pltpu.SemaphoreType.DMA,
        ],
    )(body)


def _row8(v):
    return jnp.broadcast_to(v.reshape(1, -1), (8, v.size))


def kernel(x, edge_index, batch, W1, as1, ad1, b1, W2, as2, ad2, b2,
           W3, as3, ad3, b3, lW, lb):
    n = x.shape[0]
    e_tot = edge_index.shape[1] + n
    unit = _NW * _K * 2
    e2 = ((e_tot + unit - 1) // unit) * unit
    epad = e2 - e_tot

    loop = jnp.arange(n, dtype=jnp.int32)
    src = jnp.concatenate(
        [edge_index[0].astype(jnp.int32), loop,
         jnp.zeros((epad,), jnp.int32)])
    dst = jnp.concatenate(
        [edge_index[1].astype(jnp.int32), loop,
         jnp.full((epad,), _TRASH, jnp.int32)])
    sd = jnp.stack([src.reshape(-1, _K), dst.reshape(-1, _K)],
                   axis=1).reshape(-1)

    xp = jnp.pad(x, ((0, _NPAD - n), (0, 0)))
    batch2d = jnp.pad(batch.astype(jnp.int32), (0, _NPAD - n),
                      constant_values=10000).reshape(_NPAD, 1)
    lwp = jnp.pad(lW, ((0, 0), (0, 128 - lW.shape[1])))
    lbp = _row8(jnp.pad(lb, (0, 128 - lb.shape[0])))

    t1, a1 = _front_call(128, 64, 8)(
        xp, W1, _row8(as1.reshape(-1)), _row8(ad1.reshape(-1)))
    p0, p1 = _edge_call(64, 8, e2)(t1, a1, sd)

    t2, a2 = _combine_front_call(64, 64, 8)(
        p0, p1, _row8(b1), W2, _row8(as2.reshape(-1)), _row8(ad2.reshape(-1)))
    q0, q1 = _edge_call(64, 8, e2)(t2, a2, sd)

    t3, a3 = _combine_front_call(64, 32, 1)(
        q0, q1, _row8(b2), W3, _row8(as3.reshape(-1)), _row8(ad3.reshape(-1)))
    r0, r1 = _edge_call(32, 1, e2)(t3, a3, sd)

    out = _final_call(32)(r0, r1, _row8(b3), batch2d, lwp, lbp)
    return out[:64, :2]
